# baseline (speedup 1.0000x reference)
import functools

import jax
import jax.numpy as jnp
from jax import lax
from jax.experimental import pallas as pl
from jax.experimental.pallas import tpu as pltpu
from jax.experimental.pallas import tpu_sc as plsc

_B, _C, _S, _H, _D = 64, 17, 80, 4, 32
_TCN, _MLPD, _LAT = 64, 256, 1024
_HD = _H * _D
_L = 16
_NB = _S // _L
_NC, _NS = 2, 16
_NW = _NC * _NS
_GROUPS = _B * _NB
_GPW = _GROUPS // _NW


def _build_sc_attention(chunk=0, nchunks=1, interpret=False):
    bs = _B // nchunks
    gpw = bs // _NW
    assert gpw * _NW == bs
    mesh = plsc.VectorSubcoreMesh(
        core_axis_name="c", subcore_axis_name="s",
        num_cores=_NC, num_subcores=_NS)

    @functools.partial(
        pl.kernel,
        out_type=jax.ShapeDtypeStruct((bs, _C, _S, _HD), jnp.float32),
        mesh=mesh,
        scratch_types=[
            pltpu.VMEM((_C, _S), jnp.float32),
            pltpu.VMEM((2 * _H, _L), jnp.float32),
            pltpu.VMEM((_C, _S, _H), jnp.float32),
        ],
        compiler_params=pltpu.CompilerParams(
            use_tc_tiling_on_sc=False, needs_layout_passes=False),
        interpret=interpret,
    )
    def sc_att(x_hbm, wsd_hbm, out_hbm, xv, wv, ov):
        wid = lax.axis_index("s") * _NC + lax.axis_index("c")
        pltpu.sync_copy(wsd_hbm, wv)

        def group_body(l, carry):
            b = wid * gpw + l
            pltpu.sync_copy(x_hbm.at[b + chunk * bs], xv)
            lane = lax.iota(jnp.int32, _L)
            for v in range(_NB):
                xs = [xv[i, pl.ds(v * _L, _L)] for i in range(_C)]
                s_idx = lane + (v * _L)
                for h in range(_H):
                    ws_h = wv[h]
                    wd_h = wv[_H + h]
                    a_s = [xs[i] * ws_h for i in range(_C)]

                    def j_body(j, c2, wd_h=wd_h, a_s=a_s, xs=xs,
                               s_idx=s_idx, h=h, v=v):
                        x_j = xv[j, pl.ds(v * _L, _L)]
                        bj = x_j * wd_h
                        denom = jnp.full((_L,), 1e-16, jnp.float32)
                        acc = jnp.zeros((_L,), jnp.float32)
                        for i in range(_C):
                            t = a_s[i] + bj
                            e = jnp.maximum(t, 0.2 * t)
                            z = jnp.exp(e)
                            denom = denom + z
                            acc = acc + z * xs[i]
                        t = x_j * wv[h] + bj
                        e = jnp.maximum(t, 0.2 * t)
                        z = jnp.exp(e)
                        denom = denom - z
                        acc = acc - z * x_j
                        s_jh = acc / denom
                        plsc.store_scatter(
                            ov,
                            [jnp.full((_L,), j, jnp.int32), s_idx,
                             jnp.full((_L,), h, jnp.int32)],
                            s_jh)
                        return c2

                    lax.fori_loop(0, _C, j_body, 0)
            pltpu.sync_copy(ov, out_hbm.at[b, :, :, pl.ds(0, _H)])
            return carry

        lax.fori_loop(0, gpw, group_body, 0)

    return sc_att


@functools.lru_cache(maxsize=None)
def _get_sc_attention(chunk=0, nchunks=1):
    return _build_sc_attention(chunk, nchunks)


def _conv_body(s_ref, g_ref, gb_ref, w_ref, cb_ref, gsc_ref, gbt_ref,
               m0_ref, m2_ref, out_ref):
    sb = s_ref[0].reshape(_C * _S, _HD)[:, 0:_H]
    lin = jnp.dot(sb, g_ref[...], preferred_element_type=jnp.float32)
    lin = lin + gb_ref[...]
    seq = jnp.where(lin > 0, lin, jnp.exp(jnp.minimum(lin, 0.0)) - 1.0)
    zrow = jnp.zeros((1, _HD), jnp.float32)
    sm1 = jnp.concatenate([zrow, seq[:-1]], axis=0) * m0_ref[...]
    sp1 = jnp.concatenate([seq[1:], zrow], axis=0) * m2_ref[...]
    conv = (jnp.dot(sm1, w_ref[0], preferred_element_type=jnp.float32)
            + jnp.dot(seq, w_ref[1], preferred_element_type=jnp.float32)
            + jnp.dot(sp1, w_ref[2], preferred_element_type=jnp.float32))
    conv = conv + cb_ref[...]
    y = jnp.maximum(conv, 0.01 * conv)
    y = y * gsc_ref[...] + gbt_ref[...]
    out_ref[0] = jnp.mean(y.reshape(_C, _S, _TCN), axis=1)


def _make_conv_call(bs=_B, interpret=False):
    return pl.pallas_call(
        _conv_body,
        grid=(bs,),
        in_specs=[
            pl.BlockSpec((1, _C, _S, _HD), lambda b: (b, 0, 0, 0)),
            pl.BlockSpec((_H, _HD), lambda b: (0, 0)),
            pl.BlockSpec((1, _HD), lambda b: (0, 0)),
            pl.BlockSpec((3, _HD, _TCN), lambda b: (0, 0, 0)),
            pl.BlockSpec((1, _TCN), lambda b: (0, 0)),
            pl.BlockSpec((1, _TCN), lambda b: (0, 0)),
            pl.BlockSpec((1, _TCN), lambda b: (0, 0)),
            pl.BlockSpec((_C * _S, 1), lambda b: (0, 0)),
            pl.BlockSpec((_C * _S, 1), lambda b: (0, 0)),
        ],
        out_specs=pl.BlockSpec((1, _C, _TCN), lambda b: (b, 0, 0)),
        out_shape=jax.ShapeDtypeStruct((bs, _C, _TCN), jnp.float32),
        interpret=interpret,
    )


_conv_call = _make_conv_call(_B // 2)


_DN_T = (((1,), (1,)), ((), ()))


def _mlp_body(ro_ref, w1_ref, b1_ref, w2_ref, b2_ref, out_ref):
    h1 = lax.dot_general(ro_ref[...], w1_ref[...], _DN_T,
                         preferred_element_type=jnp.float32) + b1_ref[...]
    h1 = jnp.maximum(h1, 0.01 * h1)
    out_ref[...] = lax.dot_general(h1, w2_ref[...], _DN_T,
                                   preferred_element_type=jnp.float32) + b2_ref[...]


def _make_mlp_call(interpret=False):
    return pl.pallas_call(
        _mlp_body,
        out_shape=jax.ShapeDtypeStruct((_B, _LAT), jnp.float32),
        interpret=interpret,
    )


_mlp_call = _make_mlp_call()


def kernel(x, gat_w, att_src, att_dst, gat_bias, conv_w, conv_b, bn_gamma,
           bn_beta, fc1_w, fc1_b, fc2_w, fc2_b, edge_index):
    del edge_index
    gw = gat_w.reshape(_H, _D)
    ws = jnp.sum(gw * att_src, axis=1)
    wd = jnp.sum(gw * att_dst, axis=1)
    wsd = jnp.broadcast_to(
        jnp.concatenate([ws, wd]).reshape(2 * _H, 1), (2 * _H, _L))
    g_mat = (jnp.eye(_H, dtype=jnp.float32)[:, :, None]
             * gw[None, :, :]).reshape(_H, _HD)
    w_taps = jnp.transpose(conv_w, (2, 1, 0))
    bscale = (bn_gamma / jnp.sqrt(1.0 + 1e-5)).reshape(1, _TCN)
    t_idx = jnp.arange(_C * _S, dtype=jnp.int32).reshape(-1, 1) % _S
    m0 = (t_idx != 0).astype(jnp.float32)
    m2 = (t_idx != _S - 1).astype(jnp.float32)
    nch = 2
    ro_parts = []
    for k in range(nch):
        sk = _get_sc_attention(k, nch)(x, wsd)
        ro_parts.append(
            _conv_call(sk, g_mat, gat_bias.reshape(1, _HD), w_taps,
                       conv_b.reshape(1, _TCN), bscale,
                       bn_beta.reshape(1, _TCN), m0, m2))
    ro = jnp.concatenate(ro_parts, axis=0).reshape(_B, _C * _TCN)
    out = _mlp_call(ro, fc1_w, fc1_b.reshape(1, _MLPD),
                    fc2_w, fc2_b.reshape(1, _LAT))
    return out


# speedup vs baseline: 1.5178x; 1.5178x over previous
import functools

import jax
import jax.numpy as jnp
from jax import lax
from jax.experimental import pallas as pl
from jax.experimental.pallas import tpu as pltpu
from jax.experimental.pallas import tpu_sc as plsc

_B, _C, _S, _H, _D = 64, 17, 80, 4, 32
_TCN, _MLPD, _LAT = 64, 256, 1024
_HD = _H * _D
_L = 16
_NB = _S // _L
_NC, _NS = 2, 16
_NW = _NC * _NS
_GROUPS = _B * _NB
_GPW = _GROUPS // _NW


def _build_sc_attention(chunk=0, nchunks=1, interpret=False):
    bs = _B // nchunks
    gpw = bs // _NW
    assert gpw * _NW == bs
    mesh = plsc.VectorSubcoreMesh(
        core_axis_name="c", subcore_axis_name="s",
        num_cores=_NC, num_subcores=_NS)

    @functools.partial(
        pl.kernel,
        out_type=jax.ShapeDtypeStruct((bs, _C, _H, 128), jnp.float32),
        mesh=mesh,
        scratch_types=[
            pltpu.VMEM((_C, _S), jnp.float32),
            pltpu.VMEM((2 * _H, _L), jnp.float32),
            pltpu.VMEM((_C, _H, 128), jnp.float32),
        ],
        compiler_params=pltpu.CompilerParams(
            use_tc_tiling_on_sc=False, needs_layout_passes=False),
        interpret=interpret,
    )
    def sc_att(x_hbm, wsd_hbm, out_hbm, xv, wv, ov):
        wid = lax.axis_index("s") * _NC + lax.axis_index("c")
        pltpu.sync_copy(wsd_hbm, wv)

        def group_body(l, carry):
            b = wid * gpw + l
            pltpu.sync_copy(x_hbm.at[b + chunk * bs], xv)
            for v in range(_NB):
                xs = [xv[i, pl.ds(v * _L, _L)] for i in range(_C)]
                for h in range(_H):
                    ws_h = wv[h]
                    wd_h = wv[_H + h]
                    a_s = [xs[i] * ws_h for i in range(_C)]

                    def j_body(j, c2, wd_h=wd_h, a_s=a_s, xs=xs,
                               h=h, v=v):
                        x_j = xv[j, pl.ds(v * _L, _L)]
                        bj = x_j * wd_h
                        denom = jnp.full((_L,), 1e-16, jnp.float32)
                        acc = jnp.zeros((_L,), jnp.float32)
                        for i in range(_C):
                            t = a_s[i] + bj
                            e = jnp.maximum(t, 0.2 * t)
                            z = jnp.exp(e)
                            denom = denom + z
                            acc = acc + z * xs[i]
                        t = x_j * wv[h] + bj
                        e = jnp.maximum(t, 0.2 * t)
                        z = jnp.exp(e)
                        denom = denom - z
                        acc = acc - z * x_j
                        ov[j, h, pl.ds(v * _L, _L)] = acc / denom
                        return c2

                    lax.fori_loop(0, _C, j_body, 0)
            pltpu.sync_copy(ov, out_hbm.at[b])
            return carry

        lax.fori_loop(0, gpw, group_body, 0)

    return sc_att


@functools.lru_cache(maxsize=None)
def _get_sc_attention(chunk=0, nchunks=1):
    return _build_sc_attention(chunk, nchunks)


_CL = _C * 128


def _conv_body(s_ref, gt_ref, gb_ref, w_ref, cb_ref, gsc_ref, gbt_ref,
               m0_ref, m2_ref, ones_ref, out_ref, lin_scr):
    tmask = lax.broadcasted_iota(jnp.int32, (1, 1, 128), 2) < _S
    s3 = jnp.where(tmask, s_ref[0], 0.0)
    gt = gt_ref[...]
    for c in range(_C):
        lin_scr[:, pl.ds(c * 128, 128)] = jnp.dot(
            gt, s3[c], preferred_element_type=jnp.float32)
    lin = lin_scr[...] + gb_ref[...]
    seq = jnp.where(lin > 0, lin, jnp.exp(jnp.minimum(lin, 0.0)) - 1.0)
    p0 = jnp.dot(w_ref[0], seq, preferred_element_type=jnp.float32)
    p1 = jnp.dot(w_ref[1], seq, preferred_element_type=jnp.float32)
    p2 = jnp.dot(w_ref[2], seq, preferred_element_type=jnp.float32)
    zcol = jnp.zeros((_TCN, 1), jnp.float32)
    conv = (jnp.concatenate([zcol, p0[:, :-1]], axis=1) * m0_ref[...]
            + p1
            + jnp.concatenate([p2[:, 1:], zcol], axis=1) * m2_ref[...])
    conv = conv + cb_ref[...]
    y = jnp.maximum(conv, 0.01 * conv)
    y = y * gsc_ref[...] + gbt_ref[...]
    out_ref[0] = lax.dot_general(ones_ref[...], y, _DN_T,
                                 preferred_element_type=jnp.float32)


def _make_conv_call(bs=_B, interpret=False):
    return pl.pallas_call(
        _conv_body,
        grid=(bs,),
        in_specs=[
            pl.BlockSpec((1, _C, _H, 128), lambda b: (b, 0, 0, 0)),
            pl.BlockSpec((_HD, _H), lambda b: (0, 0)),
            pl.BlockSpec((_HD, 1), lambda b: (0, 0)),
            pl.BlockSpec((3, _TCN, _HD), lambda b: (0, 0, 0)),
            pl.BlockSpec((_TCN, 1), lambda b: (0, 0)),
            pl.BlockSpec((_TCN, 1), lambda b: (0, 0)),
            pl.BlockSpec((_TCN, 1), lambda b: (0, 0)),
            pl.BlockSpec((1, _CL), lambda b: (0, 0)),
            pl.BlockSpec((1, _CL), lambda b: (0, 0)),
            pl.BlockSpec((_C, _CL), lambda b: (0, 0)),
        ],
        out_specs=pl.BlockSpec((1, _C, _TCN), lambda b: (b, 0, 0)),
        out_shape=jax.ShapeDtypeStruct((bs, _C, _TCN), jnp.float32),
        scratch_shapes=[pltpu.VMEM((_HD, _CL), jnp.float32)],
        interpret=interpret,
    )


_conv_call = _make_conv_call(_B // 2)


_DN_T = (((1,), (1,)), ((), ()))


def _mlp_body(ro_ref, w1_ref, b1_ref, w2_ref, b2_ref, out_ref):
    h1 = lax.dot_general(ro_ref[...], w1_ref[...], _DN_T,
                         preferred_element_type=jnp.float32) + b1_ref[...]
    h1 = jnp.maximum(h1, 0.01 * h1)
    out_ref[...] = lax.dot_general(h1, w2_ref[...], _DN_T,
                                   preferred_element_type=jnp.float32) + b2_ref[...]


def _make_mlp_call(interpret=False):
    return pl.pallas_call(
        _mlp_body,
        out_shape=jax.ShapeDtypeStruct((_B, _LAT), jnp.float32),
        interpret=interpret,
    )


_mlp_call = _make_mlp_call()


def kernel(x, gat_w, att_src, att_dst, gat_bias, conv_w, conv_b, bn_gamma,
           bn_beta, fc1_w, fc1_b, fc2_w, fc2_b, edge_index):
    del edge_index
    gw = gat_w.reshape(_H, _D)
    ws = jnp.sum(gw * att_src, axis=1)
    wd = jnp.sum(gw * att_dst, axis=1)
    wsd = jnp.broadcast_to(
        jnp.concatenate([ws, wd]).reshape(2 * _H, 1), (2 * _H, _L))
    gt = (jnp.eye(_H, dtype=jnp.float32)[:, :, None]
          * gw[None, :, :]).reshape(_H, _HD).T
    w_taps = jnp.transpose(conv_w, (2, 0, 1))
    bscale = (bn_gamma / jnp.sqrt(1.0 + 1e-5)).reshape(_TCN, 1)
    lam = jnp.arange(_CL, dtype=jnp.int32).reshape(1, _CL)
    t_l = lam % 128
    valid = t_l < _S
    m0 = ((t_l != 0) & valid).astype(jnp.float32)
    m2 = ((t_l != _S - 1) & valid).astype(jnp.float32)
    cols = jnp.arange(_C, dtype=jnp.int32).reshape(_C, 1)
    ones_m = (((lam // 128) == cols) & valid).astype(jnp.float32) / _S
    nch = 2
    ro_parts = []
    for k in range(nch):
        sk = _get_sc_attention(k, nch)(x, wsd)
        ro_parts.append(
            _conv_call(sk, gt, gat_bias.reshape(_HD, 1), w_taps,
                       conv_b.reshape(_TCN, 1), bscale,
                       bn_beta.reshape(_TCN, 1), m0, m2, ones_m))
    ro = jnp.concatenate(ro_parts, axis=0).reshape(_B, _C * _TCN)
    out = _mlp_call(ro, fc1_w, fc1_b.reshape(1, _MLPD),
                    fc2_w, fc2_b.reshape(1, _LAT))
    return out
